# baseline (device time: 92200 ns/iter reference)
import jax
import jax.numpy as jnp
from jax import lax
from jax.experimental import pallas as pl
from jax.experimental.pallas import tpu as pltpu

N_DEV = 4
N_SUB = 2

F_DIAG_LO, F_DIAG_HI, F_DIR_LO, F_DIR_HI, F_REL_LO, F_REL_HI = range(6)


def kernel(x, w_mat, scale_x, scale_w):
    m_total, k = x.shape
    n = w_mat.shape[1]
    m_per = m_total // N_DEV
    n_half = n // 2
    m_sub = m_per // N_SUB

    def body(x_ref, w_ref, sx_ref, sw_ref, out_ref,
             x8_ref, w8_ref,
             sd_lo, sd_hi, dir_lo, dir_hi,
             din_lo, din_hi, rin_lo, rin_hi, xin_lo, xin_hi,
             send_sems, recv_sems):
        my = lax.axis_index("i")
        left = lax.rem(my + N_DEV - 1, N_DEV)
        right = lax.rem(my + 1, N_DEV)

        barrier_sem = pltpu.get_barrier_semaphore()
        for nbr in (left, right):
            pl.semaphore_signal(barrier_sem, inc=1, device_id=(nbr,),
                                device_id_type=pl.DeviceIdType.MESH)

        def rows(b):
            return pl.ds(b * m_sub, m_sub)

        def cast_rows(c, b0, nb):
            r0 = c * m_per + b0 * m_sub
            x8_ref[pl.ds(r0, nb * m_sub), :] = x_ref[
                pl.ds(r0, nb * m_sub), :].astype(jnp.float8_e4m3fn)

        def partial(c, lo, b0, nb):
            xs = x8_ref[pl.ds(c * m_per + b0 * m_sub, nb * m_sub), :]
            ws = w8_ref[:, 0:n_half] if lo else w8_ref[:, n_half:n]
            return lax.dot_general(
                xs, ws,
                dimension_numbers=(((1,), (0,)), ((), ())),
                preferred_element_type=jnp.float32,
            )

        def xfer(src, dst, flow, b, to_right):
            return pltpu.make_async_remote_copy(
                src_ref=src.at[rows(b)], dst_ref=dst.at[rows(b)],
                send_sem=send_sems.at[flow, b],
                recv_sem=recv_sems.at[flow, b],
                device_id=(right,) if to_right else (left,),
                device_id_type=pl.DeviceIdType.MESH)

        def silu_store(acc, lo, b):
            y = acc * (sx_ref[0] * sw_ref[0])
            o = y * (1.0 / (1.0 + jnp.exp(-y)))
            if lo:
                out_ref[rows(b), 0:n_half] = o
            else:
                out_ref[rows(b), n_half:n] = o

        c_m1 = lax.rem(my + N_DEV - 1, N_DEV)
        c_p1 = lax.rem(my + 1, N_DEV)
        c_p2 = lax.rem(my + 2, N_DEV)

        w8_ref[:, 0:n_half] = w_ref[:, 0:n_half].astype(jnp.float8_e4m3fn)
        cast_rows(c_p2, 0, 1)
        sd_lo[rows(0)] = partial(c_p2, True, 0, 1).astype(jnp.bfloat16)
        pl.semaphore_wait(barrier_sem, 2)
        diag_sends = [xfer(sd_lo, din_lo, F_DIAG_LO, 0, True)]
        diag_sends[0].start()

        w8_ref[:, n_half:n] = w_ref[:, n_half:n].astype(jnp.float8_e4m3fn)
        sd_hi[rows(0)] = partial(c_p2, False, 0, 1).astype(jnp.bfloat16)
        diag_sends.append(xfer(sd_hi, din_hi, F_DIAG_HI, 0, False))
        diag_sends[1].start()

        for b in range(1, N_SUB):
            cast_rows(c_p2, b, 1)
            sd_lo[rows(b)] = partial(c_p2, True, b, 1).astype(jnp.bfloat16)
            d = xfer(sd_lo, din_lo, F_DIAG_LO, b, True)
            d.start()
            diag_sends.append(d)
            sd_hi[rows(b)] = partial(c_p2, False, b, 1).astype(jnp.bfloat16)
            d = xfer(sd_hi, din_hi, F_DIAG_HI, b, False)
            d.start()
            diag_sends.append(d)

        other_sends = []
        cast_rows(c_p1, 0, N_SUB)
        dir_hi[...] = partial(c_p1, False, 0, N_SUB).astype(jnp.bfloat16)
        for b in range(N_SUB):
            d = xfer(dir_hi, xin_hi, F_DIR_HI, b, True)
            d.start()
            other_sends.append(d)
        cast_rows(c_m1, 0, N_SUB)
        dir_lo[...] = partial(c_m1, True, 0, N_SUB).astype(jnp.bfloat16)
        for b in range(N_SUB):
            d = xfer(dir_lo, xin_lo, F_DIR_LO, b, False)
            d.start()
            other_sends.append(d)

        for b in range(N_SUB):
            xfer(sd_lo, din_lo, F_DIAG_LO, b, False).wait_recv()
            din_lo[rows(b)] = din_lo[rows(b)] + partial(
                c_p1, True, b, 1).astype(jnp.bfloat16)
            d = xfer(din_lo, rin_lo, F_REL_LO, b, True)
            d.start()
            other_sends.append(d)
            xfer(sd_hi, din_hi, F_DIAG_HI, b, True).wait_recv()
            din_hi[rows(b)] = din_hi[rows(b)] + partial(
                c_m1, False, b, 1).astype(jnp.bfloat16)
            d = xfer(din_hi, rin_hi, F_REL_HI, b, False)
            d.start()
            other_sends.append(d)

        for d in diag_sends:
            d.wait_send()
        cast_rows(my, 0, N_SUB)
        sd_lo[...] = partial(my, True, 0, N_SUB).astype(jnp.bfloat16)
        sd_hi[...] = partial(my, False, 0, N_SUB).astype(jnp.bfloat16)

        for b in range(N_SUB):
            xfer(din_lo, rin_lo, F_REL_LO, b, False).wait_recv()
            xfer(dir_lo, xin_lo, F_DIR_LO, b, True).wait_recv()
            silu_store(rin_lo[rows(b)].astype(jnp.float32)
                       + xin_lo[rows(b)].astype(jnp.float32)
                       + sd_lo[rows(b)].astype(jnp.float32), True, b)
            xfer(din_hi, rin_hi, F_REL_HI, b, True).wait_recv()
            xfer(dir_hi, xin_hi, F_DIR_HI, b, False).wait_recv()
            silu_store(rin_hi[rows(b)].astype(jnp.float32)
                       + xin_hi[rows(b)].astype(jnp.float32)
                       + sd_hi[rows(b)].astype(jnp.float32), False, b)

        for d in other_sends:
            d.wait_send()

    return pl.pallas_call(
        body,
        out_shape=jax.ShapeDtypeStruct((m_per, n), jnp.float32),
        in_specs=[
            pl.BlockSpec(memory_space=pltpu.VMEM),
            pl.BlockSpec(memory_space=pltpu.VMEM),
            pl.BlockSpec(memory_space=pltpu.SMEM),
            pl.BlockSpec(memory_space=pltpu.SMEM),
        ],
        out_specs=pl.BlockSpec(memory_space=pltpu.VMEM),
        scratch_shapes=[
            pltpu.VMEM((m_total, k), jnp.float8_e4m3fn),
            pltpu.VMEM((k, n), jnp.float8_e4m3fn),
            pltpu.VMEM((m_per, n_half), jnp.bfloat16),
            pltpu.VMEM((m_per, n_half), jnp.bfloat16),
            pltpu.VMEM((m_per, n_half), jnp.bfloat16),
            pltpu.VMEM((m_per, n_half), jnp.bfloat16),
            pltpu.VMEM((m_per, n_half), jnp.bfloat16),
            pltpu.VMEM((m_per, n_half), jnp.bfloat16),
            pltpu.VMEM((m_per, n_half), jnp.bfloat16),
            pltpu.VMEM((m_per, n_half), jnp.bfloat16),
            pltpu.VMEM((m_per, n_half), jnp.bfloat16),
            pltpu.VMEM((m_per, n_half), jnp.bfloat16),
            pltpu.SemaphoreType.DMA((6, N_SUB)),
            pltpu.SemaphoreType.DMA((6, N_SUB)),
        ],
        compiler_params=pltpu.CompilerParams(
            collective_id=0,
            vmem_limit_bytes=120 * 1024 * 1024,
        ),
    )(x, w_mat, scale_x, scale_w)


# device time: 91557 ns/iter; 1.0070x vs baseline; 1.0070x over previous
import jax
import jax.numpy as jnp
from jax import lax
from jax.experimental import pallas as pl
from jax.experimental.pallas import tpu as pltpu

N_DEV = 4
N_SUB = 4


def kernel(x, w_mat, scale_x, scale_w):
    m_total, k = x.shape
    n = w_mat.shape[1]
    m_per = m_total // N_DEV
    n_half = n // 2
    m_sub = m_per // N_SUB

    def body(x_ref, w_ref, sx_ref, sw_ref, out_ref,
             x8_ref, w8_ref, send_r_ref, send_l_ref, recv_r_ref, recv_l_ref,
             send_sems_r, recv_sems_r, send_sems_l, recv_sems_l):
        my = lax.axis_index("i")
        left = lax.rem(my + N_DEV - 1, N_DEV)
        right = lax.rem(my + 1, N_DEV)

        barrier_sem = pltpu.get_barrier_semaphore()
        for nbr in (left, right):
            pl.semaphore_signal(barrier_sem, inc=1, device_id=(nbr,),
                                device_id_type=pl.DeviceIdType.MESH)

        def rows(b):
            return pl.ds(b * m_sub, m_sub)

        def cast_rows(c, b0, nb):
            r0 = c * m_per + b0 * m_sub
            x8_ref[pl.ds(r0, nb * m_sub), :] = x_ref[
                pl.ds(r0, nb * m_sub), :].astype(jnp.float8_e4m3fn)

        def partial(c, lo, b0, nb):
            xs = x8_ref[pl.ds(c * m_per + b0 * m_sub, nb * m_sub), :]
            ws = w8_ref[:, 0:n_half] if lo else w8_ref[:, n_half:n]
            return lax.dot_general(
                xs, ws,
                dimension_numbers=(((1,), (0,)), ((), ())),
                preferred_element_type=jnp.float32,
            )

        def rdma(dirn, s, b):
            if dirn == "r":
                return pltpu.make_async_remote_copy(
                    src_ref=send_r_ref.at[s % 2, rows(b)],
                    dst_ref=recv_r_ref.at[s, rows(b)],
                    send_sem=send_sems_r.at[s, b],
                    recv_sem=recv_sems_r.at[s, b],
                    device_id=(right,), device_id_type=pl.DeviceIdType.MESH)
            return pltpu.make_async_remote_copy(
                src_ref=send_l_ref.at[s % 2, rows(b)],
                dst_ref=recv_l_ref.at[s, rows(b)],
                send_sem=send_sems_l.at[s, b],
                recv_sem=recv_sems_l.at[s, b],
                device_id=(left,), device_id_type=pl.DeviceIdType.MESH)

        def silu_store(acc, lo, b):
            y = acc * (sx_ref[0] * sw_ref[0])
            o = y * (1.0 / (1.0 + jnp.exp(-y)))
            if lo:
                out_ref[rows(b), 0:n_half] = o
            else:
                out_ref[rows(b), n_half:n] = o

        c_m1 = lax.rem(my + N_DEV - 1, N_DEV)
        c_p1 = lax.rem(my + 1, N_DEV)
        c_p2 = lax.rem(my + 2, N_DEV)

        prev_r, prev_l = [], []
        w8_ref[:, 0:n_half] = w_ref[:, 0:n_half].astype(jnp.float8_e4m3fn)
        for b in range(N_SUB):
            cast_rows(c_m1, b, 1)
            send_r_ref[0, rows(b)] = partial(c_m1, True, b, 1).astype(
                jnp.bfloat16)
            if b == 0:
                pl.semaphore_wait(barrier_sem, 2)
            d = rdma("r", 0, b)
            d.start()
            prev_r.append(d)
            if b == 0:
                w8_ref[:, n_half:n] = w_ref[:, n_half:n].astype(
                    jnp.float8_e4m3fn)
            cast_rows(c_p1, b, 1)
            send_l_ref[0, rows(b)] = partial(c_p1, False, b, 1).astype(
                jnp.bfloat16)
            d = rdma("l", 0, b)
            d.start()
            prev_l.append(d)

        for s in (1, 2):
            slot = s % 2
            if s == 1:
                cast_rows(c_p2, 0, N_SUB)
                cr, cl = c_p2, c_p2
            else:
                cast_rows(my, 0, N_SUB)
                cr, cl = c_p1, c_m1
            send_r_ref[slot] = partial(cr, True, 0, N_SUB).astype(
                jnp.bfloat16)
            send_l_ref[slot] = partial(cl, False, 0, N_SUB).astype(
                jnp.bfloat16)
            cur_r, cur_l = [], []
            for b in range(N_SUB):
                prev_r[b].wait()
                send_r_ref[slot, rows(b)] = (send_r_ref[slot, rows(b)]
                                             + recv_r_ref[s - 1, rows(b)])
                d = rdma("r", s, b)
                d.start()
                cur_r.append(d)
                prev_l[b].wait()
                send_l_ref[slot, rows(b)] = (send_l_ref[slot, rows(b)]
                                             + recv_l_ref[s - 1, rows(b)])
                d = rdma("l", s, b)
                d.start()
                cur_l.append(d)
            prev_r, prev_l = cur_r, cur_l

        send_r_ref[1] = partial(my, True, 0, N_SUB).astype(jnp.bfloat16)
        send_l_ref[1] = partial(my, False, 0, N_SUB).astype(jnp.bfloat16)
        for b in range(N_SUB):
            prev_r[b].wait()
            silu_store(send_r_ref[1, rows(b)].astype(jnp.float32)
                       + recv_r_ref[2, rows(b)].astype(jnp.float32), True, b)
            prev_l[b].wait()
            silu_store(send_l_ref[1, rows(b)].astype(jnp.float32)
                       + recv_l_ref[2, rows(b)].astype(jnp.float32), False, b)

    return pl.pallas_call(
        body,
        out_shape=jax.ShapeDtypeStruct((m_per, n), jnp.float32),
        in_specs=[
            pl.BlockSpec(memory_space=pltpu.VMEM),
            pl.BlockSpec(memory_space=pltpu.VMEM),
            pl.BlockSpec(memory_space=pltpu.SMEM),
            pl.BlockSpec(memory_space=pltpu.SMEM),
        ],
        out_specs=pl.BlockSpec(memory_space=pltpu.VMEM),
        scratch_shapes=[
            pltpu.VMEM((m_total, k), jnp.float8_e4m3fn),
            pltpu.VMEM((k, n), jnp.float8_e4m3fn),
            pltpu.VMEM((2, m_per, n_half), jnp.bfloat16),
            pltpu.VMEM((2, m_per, n_half), jnp.bfloat16),
            pltpu.VMEM((N_DEV - 1, m_per, n_half), jnp.bfloat16),
            pltpu.VMEM((N_DEV - 1, m_per, n_half), jnp.bfloat16),
            pltpu.SemaphoreType.DMA((N_DEV - 1, N_SUB)),
            pltpu.SemaphoreType.DMA((N_DEV - 1, N_SUB)),
            pltpu.SemaphoreType.DMA((N_DEV - 1, N_SUB)),
            pltpu.SemaphoreType.DMA((N_DEV - 1, N_SUB)),
        ],
        compiler_params=pltpu.CompilerParams(
            collective_id=0,
            vmem_limit_bytes=120 * 1024 * 1024,
        ),
    )(x, w_mat, scale_x, scale_w)
